# Initial kernel scaffold; baseline (speedup 1.0000x reference)
import jax
import jax.numpy as jnp
from jax.experimental import pallas as pl

_HEADS = 4


def _final_body(x_ref, b_ref, o_ref):
    v = x_ref[...] + b_ref[...]
    m = jnp.max(v, axis=-1, keepdims=True)
    e = jnp.exp(v - m)
    o_ref[...] = v - m - jnp.log(jnp.sum(e, axis=-1, keepdims=True))


def kernel(x, edge_index, edge_attr, W_sage_l, W_sage_r, b_sage, W_tgat, att_src, att_dst, W_edge, att_edge, b_tgat):
    N, F = x.shape
    E = edge_attr.shape[0]
    OUT = b_tgat.shape[0]
    src = edge_index[0]
    dst = edge_index[1]
    msg = jnp.take(x, src, axis=0)
    agg = jax.ops.segment_sum(msg, dst, num_segments=N)
    deg = jax.ops.segment_sum(jnp.ones((E,), dtype=x.dtype), dst, num_segments=N)
    mean = agg / jnp.maximum(deg, 1.0)[:, None]
    h = mean @ W_sage_l + x @ W_sage_r + b_sage
    h = h / (jnp.linalg.norm(h, axis=-1, keepdims=True) + 1e-12)
    h = jax.nn.relu(h)
    hh = (h @ W_tgat).reshape(N, _HEADS, OUT)
    a_s = jnp.sum(hh * att_src[None, :, :], axis=-1)
    a_d = jnp.sum(hh * att_dst[None, :, :], axis=-1)
    eh = (edge_attr @ W_edge).reshape(E, _HEADS, OUT)
    a_e = jnp.sum(eh * att_edge[None, :, :], axis=-1)
    logits = jax.nn.leaky_relu(a_s[src] + a_d[dst] + a_e, negative_slope=0.2)
    seg_max = jax.ops.segment_max(logits, dst, num_segments=N)
    seg_max = jnp.where(jnp.isfinite(seg_max), seg_max, 0.0)
    num = jnp.exp(logits - seg_max[dst])
    den = jax.ops.segment_sum(num, dst, num_segments=N)
    alpha = num / (den[dst] + 1e-16)
    m = alpha[:, :, None] * jnp.take(hh, src, axis=0)
    out = jax.ops.segment_sum(m, dst, num_segments=N)
    out = jnp.mean(out, axis=1)

    blk = 1000
    return pl.pallas_call(
        _final_body,
        out_shape=jax.ShapeDtypeStruct((N, OUT), jnp.float32),
        grid=(N // blk,),
        in_specs=[
            pl.BlockSpec((blk, OUT), lambda i: (i, 0)),
            pl.BlockSpec((1, OUT), lambda i: (0, 0)),
        ],
        out_specs=pl.BlockSpec((blk, OUT), lambda i: (i, 0)),
    )(out, b_tgat.reshape(1, OUT))


# speedup vs baseline: 6.1044x; 6.1044x over previous
import functools

import jax
import jax.numpy as jnp
from jax import lax
from jax.experimental import pallas as pl
from jax.experimental.pallas import tpu as pltpu
from jax.experimental.pallas import tpu_sc as plsc

_N = 10000
_E = 160000
_NP = 10240
_EP = 163840
_F = 256
_OUT = 128
_HEADS = 4
_NS = 16
_NROW = _NP // _NS
_EA = _EP // _NS
_BA = 512
_BC = 256
_W = 64

_f32 = jnp.float32
_i32 = jnp.int32

_MESH = plsc.VectorSubcoreMesh(core_axis_name="c", subcore_axis_name="s")


def _sage_body(xq0, xq1, xq2, xq3, srcp, dstp, zrow, zvec, onesb,
               agg_q0, agg_q1, agg_q2, agg_q3, deg,
               idx_s, idx_d, rows, ones_v, sem, agg_sh, deg_sh):
    c = lax.axis_index("c")
    s = lax.axis_index("s")
    base_n = s * _NROW
    ebase = s * _EA
    pltpu.sync_copy(onesb, ones_v)

    def quarter_pass(xref, agg_out, with_deg):
        pltpu.sync_copy(zrow, agg_sh.at[pl.ds(base_n, _NROW)])
        if with_deg:
            pltpu.sync_copy(zvec, deg_sh.at[pl.ds(base_n, _NROW)])
        plsc.subcore_barrier()

        @pl.loop(0, _EA // _BA)
        def _batch(i):
            b = ebase + i * _BA
            pltpu.sync_copy(srcp.at[pl.ds(b, _BA)], idx_s)
            pltpu.sync_copy(dstp.at[pl.ds(b, _BA)], idx_d)
            pltpu.async_copy(xref.at[idx_s], rows, sem).wait()
            pltpu.sync_copy(rows, agg_sh.at[idx_d], add=True)
            if with_deg:
                pltpu.sync_copy(ones_v, deg_sh.at[idx_d], add=True)

        plsc.subcore_barrier()
        pltpu.sync_copy(agg_sh.at[pl.ds(base_n, _NROW)], agg_out.at[pl.ds(base_n, _NROW)])
        if with_deg:
            pltpu.sync_copy(deg_sh.at[pl.ds(base_n, _NROW)], deg.at[pl.ds(base_n, _NROW)])
        plsc.subcore_barrier()

    @pl.when(c == 0)
    def _():
        quarter_pass(xq0, agg_q0, True)
        quarter_pass(xq1, agg_q1, False)

    @pl.when(c == 1)
    def _():
        quarter_pass(xq2, agg_q2, False)
        quarter_pass(xq3, agg_q3, False)


_sage_call = functools.partial(
    pl.kernel,
    out_type=[jax.ShapeDtypeStruct((_NP, _W), _f32)] * 4
    + [jax.ShapeDtypeStruct((_NP,), _f32)],
    mesh=_MESH,
    compiler_params=pltpu.CompilerParams(use_tc_tiling_on_sc=False, needs_layout_passes=False),
    scratch_types=[
        pltpu.VMEM((_BA,), _i32),
        pltpu.VMEM((_BA,), _i32),
        pltpu.VMEM((_BA, _W), _f32),
        pltpu.VMEM((_BA,), _f32),
        pltpu.SemaphoreType.DMA,
        pltpu.VMEM_SHARED((_NP, _W), _f32),
        pltpu.VMEM_SHARED((_NP,), _f32),
    ],
)(_sage_body)


_EH = _EP // 32
_WQ = 32


def _attn_body(t0, t1, t2, t3, asd, ae_h, sd,
               o00, o01, o10, o11, o20, o21, o30, o31, d0, d1,
               asd_v, sd_v, idx_s, idx_d, ae_v, num_v, rows, sem, out_sh,
               den_sh):
    c = lax.axis_index("c")
    s = lax.axis_index("s")
    base_n = s * _NROW
    ebase = (c * _NS + s) * _EH
    pltpu.sync_copy(asd, asd_v)

    def quarter_pass(q, tq, oq0, oq1, with_den):
        @pl.loop(0, _BC)
        def _z(e):
            zv = jnp.zeros((16,), _f32)
            for j in range(_WQ // 16):
                rows[e, pl.ds(j * 16, 16)] = zv

        @pl.loop(0, _BC // 16)
        def _zn(k):
            num_v[pl.ds(k * 16, 16)] = jnp.zeros((16,), _f32)

        pltpu.sync_copy(rows, out_sh.at[pl.ds(base_n, _BC)])
        pltpu.sync_copy(rows, out_sh.at[pl.ds(base_n + _BC, _BC)])
        pltpu.sync_copy(rows.at[pl.ds(0, _NROW - 2 * _BC)],
                        out_sh.at[pl.ds(base_n + 2 * _BC, _NROW - 2 * _BC)])
        if with_den:
            pltpu.sync_copy(num_v, den_sh.at[pl.ds(base_n, _BC)])
            pltpu.sync_copy(num_v, den_sh.at[pl.ds(base_n + _BC, _BC)])
            pltpu.sync_copy(num_v.at[pl.ds(0, _NROW - 2 * _BC)],
                            den_sh.at[pl.ds(base_n + 2 * _BC, _NROW - 2 * _BC)])
        plsc.subcore_barrier()

        @pl.loop(0, _EH // _BC)
        def _batch(i):
            b = ebase + i * _BC
            pltpu.sync_copy(sd.at[pl.ds(b, _BC)], sd_v)
            pltpu.sync_copy(ae_h.at[pl.ds(b, _BC)], ae_v)

            @pl.loop(0, _BC // 16)
            def _unpack(k):
                sl = pl.ds(k * 16, 16)
                v = sd_v[sl]
                idx_s[sl] = v & 16383
                idx_d[sl] = v >> 14

            gat = pltpu.async_copy(tq.at[idx_s], rows, sem)

            @pl.loop(0, _BC // 16)
            def _num(k):
                sl = pl.ds(k * 16, 16)
                sv = idx_s[sl]
                dv = idx_d[sl]
                a_s = plsc.load_gather(asd_v, [sv * 8])
                a_d = plsc.load_gather(asd_v, [dv * 8 + 4])
                lg = a_s + a_d + ae_v[sl]
                lg = jnp.where(lg > 0.0, lg, lg * 0.2)
                num_v[sl] = jnp.exp(lg)

            gat.wait()

            @pl.loop(0, _BC, unroll=4)
            def _scale(e):
                w = plsc.load_gather(num_v, [jnp.full((16,), e, _i32)])
                for j in range(_WQ // 16):
                    sl = pl.ds(j * 16, 16)
                    rows[e, sl] = rows[e, sl] * w

            pltpu.sync_copy(rows, out_sh.at[idx_d], add=True)
            if with_den:
                pltpu.sync_copy(num_v, den_sh.at[idx_d], add=True)

        plsc.subcore_barrier()

        @pl.when(c == 0)
        def _():
            pltpu.sync_copy(out_sh.at[pl.ds(base_n, _NROW)],
                            oq0.at[pl.ds(base_n, _NROW)])
            if with_den:
                pltpu.sync_copy(den_sh.at[pl.ds(base_n, _NROW)],
                                d0.at[pl.ds(base_n, _NROW)])

        @pl.when(c == 1)
        def _():
            pltpu.sync_copy(out_sh.at[pl.ds(base_n, _NROW)],
                            oq1.at[pl.ds(base_n, _NROW)])
            if with_den:
                pltpu.sync_copy(den_sh.at[pl.ds(base_n, _NROW)],
                                d1.at[pl.ds(base_n, _NROW)])

        plsc.subcore_barrier()

    quarter_pass(0, t0, o00, o01, True)
    quarter_pass(1, t1, o10, o11, False)
    quarter_pass(2, t2, o20, o21, False)
    quarter_pass(3, t3, o30, o31, False)


_attn_call = functools.partial(
    pl.kernel,
    out_type=[jax.ShapeDtypeStruct((_NP, _WQ), _f32)] * 8
    + [jax.ShapeDtypeStruct((_NP,), _f32)] * 2,
    mesh=_MESH,
    compiler_params=pltpu.CompilerParams(use_tc_tiling_on_sc=False,
                                         needs_layout_passes=False),
    scratch_types=[
        pltpu.VMEM((_NP * 8,), _f32),
        pltpu.VMEM((_BC,), _i32),
        pltpu.VMEM((_BC,), _i32),
        pltpu.VMEM((_BC,), _i32),
        pltpu.VMEM((_BC,), _f32),
        pltpu.VMEM((_BC,), _f32),
        pltpu.VMEM((_BC, _WQ), _f32),
        pltpu.SemaphoreType.DMA,
        pltpu.VMEM_SHARED((_NP, _WQ), _f32),
        pltpu.VMEM_SHARED((_NP,), _f32),
    ],
)(_attn_body)


def _b1_body(*refs):
    (agg0, agg1, agg2, agg3, deg, x, wl0, wl1, wl2, wl3, wr, bsg, wt,
     bsd) = refs[:14]
    outs = refs[14:30]
    asd = refs[30]
    dm = jnp.maximum(deg[...], 1.0)
    h = ((agg0[...] / dm) @ wl0[...] + (agg1[...] / dm) @ wl1[...]
         + (agg2[...] / dm) @ wl2[...] + (agg3[...] / dm) @ wl3[...])
    h = h + x[...] @ wr[...] + bsg[...]
    h = h / (jnp.sqrt(jnp.sum(h * h, axis=-1, keepdims=True)) + 1e-12)
    h = jnp.maximum(h, 0.0)
    hh = h @ wt[...]
    asd[...] = h @ bsd[...]
    for k in range(16):
        outs[k][...] = hh[:, k * 32:(k + 1) * 32]


def _b1_call(aggs, deg2, xp, wls, wr, bsg, wt, bsd):
    R = 512
    G = _NP // R
    full = lambda shape: pl.BlockSpec(shape, lambda i: (0, 0))
    row = lambda w: pl.BlockSpec((R, w), lambda i: (i, 0))
    return pl.pallas_call(
        _b1_body,
        grid=(G,),
        in_specs=[row(_W)] * 4 + [pl.BlockSpec((R, 1), lambda i: (i, 0)), row(256)]
        + [full((_W, 256))] * 4
        + [full((256, 256)), full((1, 256)), full((256, 512)), full((256, 8))],
        out_specs=[row(_WQ)] * 16 + [row(8)],
        out_shape=[jax.ShapeDtypeStruct((_NP, _WQ), _f32)] * 16
        + [jax.ShapeDtypeStruct((_NP, 8), _f32)],
    )(*aggs, deg2, xp, *wls, wr, bsg, wt, bsd)


def _b2_body(a8, ea, ae):
    ae[...] = a8[...] @ ea[...]


def _b2_call(a8, ea_t):
    B = 4096
    return pl.pallas_call(
        _b2_body,
        grid=(_EP // B,),
        in_specs=[
            pl.BlockSpec((8, 16), lambda i: (0, 0)),
            pl.BlockSpec((16, B), lambda i: (0, i)),
        ],
        out_specs=pl.BlockSpec((8, B), lambda i: (0, i)),
        out_shape=jax.ShapeDtypeStruct((8, _EP), _f32),
    )(a8, ea_t)


def _d_body(*refs):
    oq = refs[:32]
    dn = refs[32:40]
    bt, out = refs[40], refs[41]
    vqs = []
    for q in range(4):
        vq = None
        for h in range(4):
            r = 1.0 / (dn[2 * h][...] + dn[2 * h + 1][...] + 1e-16)
            t = (oq[h * 8 + q * 2][...] + oq[h * 8 + q * 2 + 1][...]) * r
            vq = t if vq is None else vq + t
        vqs.append(vq)
    v = jnp.concatenate(vqs, axis=1) * 0.25 + bt[...]
    mx = jnp.max(v, axis=-1, keepdims=True)
    e = jnp.exp(v - mx)
    out[...] = v - mx - jnp.log(jnp.sum(e, axis=-1, keepdims=True))


def _d_call(oqs, dens, bt):
    R = 256
    row = lambda w: pl.BlockSpec((R, w), lambda i: (i, 0))
    return pl.pallas_call(
        _d_body,
        grid=(_NP // R,),
        in_specs=[row(_WQ)] * 32 + [pl.BlockSpec((R, 1), lambda i: (i, 0))] * 8
        + [pl.BlockSpec((1, 128), lambda i: (0, 0))],
        out_specs=row(128),
        out_shape=jax.ShapeDtypeStruct((_NP, 128), _f32),
    )(*oqs, *dens, bt)


def kernel(x, edge_index, edge_attr, W_sage_l, W_sage_r, b_sage, W_tgat,
           att_src, att_dst, W_edge, att_edge, b_tgat):
    src = edge_index[0].astype(_i32)
    dst = edge_index[1].astype(_i32)
    pad = jnp.full((_EP - _E,), _N, _i32)
    srcp = jnp.concatenate([src, pad])
    dstp = jnp.concatenate([dst, pad])
    xp = jnp.pad(x, ((0, _NP - _N), (0, 0)))
    xqs = [xp[:, i * _W:(i + 1) * _W] for i in range(4)]
    zrow = jnp.zeros((_NROW, _W), _f32)
    zvec = jnp.zeros((_NROW,), _f32)
    onesb = jnp.ones((_BA,), _f32)

    *aggs, deg = _sage_call(*xqs, srcp, dstp, zrow, zvec, onesb)

    wt3 = W_tgat.reshape(_F, _HEADS, _OUT)
    bsd = jnp.concatenate([
        jnp.einsum("kho,ho->kh", wt3, att_src),
        jnp.einsum("kho,ho->kh", wt3, att_dst),
    ], axis=1)
    wls = [W_sage_l[i * _W:(i + 1) * _W] for i in range(4)]
    *hhs, asd = _b1_call(aggs, deg.reshape(_NP, 1), xp, wls, W_sage_r,
                         b_sage.reshape(1, _F), W_tgat, bsd)

    a8 = jnp.pad(
        jnp.einsum("eho,ho->eh", W_edge.reshape(16, _HEADS, _OUT), att_edge).T,
        ((0, 4), (0, 0)))
    ea_t = jnp.pad(edge_attr, ((0, _EP - _E), (0, 0))).T
    ae = _b2_call(a8, ea_t)

    sdp = dstp * 16384 + srcp
    asd_pad = jnp.pad(asd.reshape(-1), (0, 8))
    oqs, dens = [], []
    for h in range(_HEADS):
        asd_h = asd_pad[h:h + _NP * 8]
        res = _attn_call(hhs[4 * h], hhs[4 * h + 1], hhs[4 * h + 2],
                         hhs[4 * h + 3], asd_h, ae[h], sdp)
        oqs += list(res[:8])
        dens += [res[8].reshape(_NP, 1), res[9].reshape(_NP, 1)]
    out = _d_call(oqs, dens, b_tgat.reshape(1, _OUT))
    return out[:_N]
